# Initial kernel scaffold; baseline (speedup 1.0000x reference)
import jax
import jax.numpy as jnp
from jax.experimental import pallas as pl


def kernel(x, params):
    raise NotImplementedError("write your pallas kernel here")


# speedup vs baseline: 24.3899x; 24.3899x over previous
import functools

import jax
import jax.numpy as jnp
import numpy as np
from jax import lax
from jax.experimental import pallas as pl
from jax.experimental.pallas import tpu as pltpu
from jax.experimental.pallas import tpu_sc as plsc

B, N, KNN, M = 8, 1024, 20, 8
EPS = 1e-5
INV_STD = np.float32(1.0 / np.sqrt(1.0 + EPS))
HI = jax.lax.Precision.HIGHEST

R = 256
NB = 128
NCORES, NSUBCORES = 2, 16
NW = NCORES * NSUBCORES
ROWS = B * N * KNN
RPW = ROWS // NW
CHUNK = 1280
NCHUNK = RPW // CHUNK


def _bf(v):
    return v.astype(jnp.bfloat16).astype(jnp.float32)


def _bfx(v):
    return lax.reduce_precision(v, exponent_bits=8, mantissa_bits=7)


def _knn_scores_kernel(xaug_row, xaug_full, xaug_t,
                       w1t2, g12, b12, w2t2, b22,
                       w1t3, g13, b13, w2t3, b23,
                       w1t4, g14, b14, w2t4, b24,
                       wconv, gc, bc,
                       fidx_ref, s2_ref, s3_ref, s4_ref, h1_ref):
    b = pl.program_id(0)
    a_r = xaug_row[0]
    full = xaug_full[0]
    xt_t = xaug_t[0]

    acc = a_r[:, 0:1] * xt_t[0:1, :]
    for d in (1, 2):
        acc = acc + a_r[:, d:d + 1] * xt_t[d:d + 1, :]
    pd = (a_r[:, 9:10] + (acc + acc)) + xt_t[4:5, :]

    ctr = (a_r[:, 0:3] + a_r[:, 3:6]) + a_r[:, 6:9]

    iota = lax.broadcasted_iota(jnp.int32, (R, N), 1)

    cur = pd
    idx_cols = []
    s2_cols, s3_cols, s4_cols = [], [], []
    for _ in range(KNN):
        v = jnp.max(cur, axis=1, keepdims=True)
        eqm = cur == v
        a = jnp.min(jnp.where(eqm, iota, jnp.int32(2**30)), axis=1,
                    keepdims=True)
        sel = iota == a
        onehot = sel.astype(jnp.float32)
        cur = jnp.where(sel, -jnp.inf, cur)
        idx_cols.append(a)
        nbrp = jnp.dot(onehot, full)
        nbr = (nbrp[:, 0:3] + nbrp[:, 3:6]) + nbrp[:, 6:9]
        diff = nbr - ctr
        xyzb = jnp.concatenate(
            [_bf(diff), _bf(nbr), jnp.zeros((R, 2), jnp.float32)], axis=1)
        for w1t, g1, b1, w2t, b2, cols in (
            (w1t2, g12, b12, w2t2, b22, s2_cols),
            (w1t3, g13, b13, w2t3, b23, s3_cols),
            (w1t4, g14, b14, w2t4, b24, s4_cols),
        ):
            z = jnp.dot(xyzb, w1t[...])
            act = jnp.maximum((z * INV_STD) * g1[...] + b1[...], 0.0)
            logits = jnp.dot(_bf(act), w2t[...]) + b2[...]
            e = jnp.exp(logits - jnp.max(logits, axis=1, keepdims=True))
            cols.append(e / jnp.sum(e, axis=1, keepdims=True))

    fidx_ref[0] = jnp.concatenate(idx_cols, axis=1) + b * N
    s2_ref[0] = jnp.concatenate(s2_cols, axis=1)
    s3_ref[0] = jnp.concatenate(s3_cols, axis=1)
    s4_ref[0] = jnp.concatenate(s4_cols, axis=1)
    zc = jnp.dot(a_r, wconv[...])
    h1_ref[0] = _bf(jnp.maximum((zc * INV_STD) * gc[...] + bc[...], 0.0))


def _knn_scores(xaug, xaug_t, sn_params, conv1_params):
    grid = (B, N // R)
    full_spec = pl.BlockSpec((1, N, 16), lambda b, r: (b, 0, 0))
    row_spec = pl.BlockSpec((1, R, 16), lambda b, r: (b, r, 0))
    t_spec = pl.BlockSpec((1, 16, N), lambda b, r: (b, 0, 0))

    def pspec(shape):
        return pl.BlockSpec(shape, lambda b, r: tuple(0 for _ in shape))

    in_specs = [row_spec, full_spec, t_spec]
    p_args = []
    for w1t, g1, b1, w2t, b2 in sn_params:
        p_args += [w1t, g1, b1, w2t, b2]
        in_specs += [pspec((8, 16)), pspec((1, 16)), pspec((1, 16)),
                     pspec((16, M)), pspec((1, M))]
    p_args += list(conv1_params)
    in_specs += [pspec((16, 64)), pspec((1, 64)), pspec((1, 64))]

    out_shape = [
        jax.ShapeDtypeStruct((B, N, KNN), jnp.int32),
        jax.ShapeDtypeStruct((B, N, KNN * M), jnp.float32),
        jax.ShapeDtypeStruct((B, N, KNN * M), jnp.float32),
        jax.ShapeDtypeStruct((B, N, KNN * M), jnp.float32),
        jax.ShapeDtypeStruct((B, N, 64), jnp.float32),
    ]
    out_specs = [
        pl.BlockSpec((1, R, KNN), lambda b, r: (b, r, 0)),
        pl.BlockSpec((1, R, KNN * M), lambda b, r: (b, r, 0)),
        pl.BlockSpec((1, R, KNN * M), lambda b, r: (b, r, 0)),
        pl.BlockSpec((1, R, KNN * M), lambda b, r: (b, r, 0)),
        pl.BlockSpec((1, R, 64), lambda b, r: (b, r, 0)),
    ]
    return pl.pallas_call(
        _knn_scores_kernel, grid=grid, in_specs=in_specs,
        out_specs=out_specs, out_shape=out_shape,
    )(xaug, xaug, xaug_t, *p_args)


def _sc_gather(table, fidx):
    mesh = plsc.VectorSubcoreMesh(core_axis_name="c", subcore_axis_name="s")

    @functools.partial(
        pl.kernel, mesh=mesh,
        compiler_params=pltpu.CompilerParams(use_tc_tiling_on_sc=False),
        out_type=jax.ShapeDtypeStruct((ROWS, 64), jnp.float32),
        scratch_types=[
            pltpu.VMEM((CHUNK,), jnp.int32),
            pltpu.VMEM((CHUNK, 64), jnp.float32),
            pltpu.SemaphoreType.DMA,
        ],
    )
    def gather_k(table_hbm, idx_hbm, out_hbm, idx_v, rows_v, sem):
        wid = lax.axis_index("s") * NCORES + lax.axis_index("c")
        base = wid * RPW
        for c in range(NCHUNK):
            off = base + c * CHUNK
            pltpu.sync_copy(idx_hbm.at[pl.ds(off, CHUNK)], idx_v)
            pltpu.async_copy(table_hbm.at[idx_v], rows_v, sem).wait()
            pltpu.sync_copy(rows_v, out_hbm.at[pl.ds(off, CHUNK)])

    return gather_k(table, fidx)


def _combine(g, s):
    parts = []
    for m in range(M):
        acc = jnp.zeros((NB, 64), jnp.float32)
        for k in range(KNN):
            acc = acc + s[:, k * M + m:k * M + m + 1] * g[:, k * 64:(k + 1) * 64]
        parts.append(acc)
    return jnp.concatenate(parts, axis=1)


def _layer_kernel(g_ref, s_ref, wr, gv, bv, out_ref):
    t = _combine(g_ref[0], s_ref[0])
    z = jnp.dot(t, wr[...], precision=HI)
    out_ref[0] = _bf(jnp.maximum((z * INV_STD) * gv[...] + bv[...], 0.0))


def _layer(g, s, wr, gv, bv, cout):
    grid = (B, N // NB)
    return pl.pallas_call(
        _layer_kernel, grid=grid,
        in_specs=[
            pl.BlockSpec((1, NB, KNN * 64), lambda b, r: (b, r, 0)),
            pl.BlockSpec((1, NB, KNN * M), lambda b, r: (b, r, 0)),
            pl.BlockSpec((M * 64, cout), lambda b, r: (0, 0)),
            pl.BlockSpec((1, cout), lambda b, r: (0, 0)),
            pl.BlockSpec((1, cout), lambda b, r: (0, 0)),
        ],
        out_specs=pl.BlockSpec((1, NB, cout), lambda b, r: (b, r, 0)),
        out_shape=jax.ShapeDtypeStruct((B, N, cout), jnp.float32),
    )(g, s, wr, gv, bv)


def _layer4_pool_kernel(g_ref, s_ref, wr, gv, bv, w5, g5, b5, out_ref):
    t = _combine(g_ref[0], s_ref[0])
    z = jnp.dot(t, wr[...], precision=HI)
    h4 = jnp.maximum((z * INV_STD) * gv[...] + bv[...], 0.0)
    z5 = jnp.dot(_bf(h4), w5[...])
    z5 = jnp.maximum((z5 * INV_STD) * g5[...] + b5[...], 0.0)
    pm = jnp.max(z5, axis=0, keepdims=True)

    @pl.when(pl.program_id(1) == 0)
    def _():
        out_ref[0] = pm

    @pl.when(pl.program_id(1) != 0)
    def _():
        out_ref[0] = jnp.maximum(out_ref[0], pm)


def _layer4_pool(g, s, wr, gv, bv, w5, g5, b5):
    grid = (B, N // NB)
    return pl.pallas_call(
        _layer4_pool_kernel, grid=grid,
        in_specs=[
            pl.BlockSpec((1, NB, KNN * 64), lambda b, r: (b, r, 0)),
            pl.BlockSpec((1, NB, KNN * M), lambda b, r: (b, r, 0)),
            pl.BlockSpec((M * 64, 128), lambda b, r: (0, 0)),
            pl.BlockSpec((1, 128), lambda b, r: (0, 0)),
            pl.BlockSpec((1, 128), lambda b, r: (0, 0)),
            pl.BlockSpec((128, 1024), lambda b, r: (0, 0)),
            pl.BlockSpec((1, 1024), lambda b, r: (0, 0)),
            pl.BlockSpec((1, 1024), lambda b, r: (0, 0)),
        ],
        out_specs=pl.BlockSpec((1, 1, 1024), lambda b, r: (b, 0, 0)),
        out_shape=jax.ShapeDtypeStruct((B, 1, 1024), jnp.float32),
    )(g, s, wr, gv, bv, w5, g5, b5).reshape(B, 1024)


def _head_kernel(p_ref, w6, g6, b6, w7, b7, out_ref):
    z = jnp.dot(_bf(p_ref[...]), w6[...])
    h = jnp.maximum((z * INV_STD) * g6[...] + b6[...], 0.0)
    out_ref[...] = jnp.dot(_bf(h), w7[...]) + b7[...]


def _head(pooled, w6, g6, b6, w7, b7):
    return pl.pallas_call(
        _head_kernel,
        out_shape=jax.ShapeDtypeStruct((B, 40), jnp.float32),
    )(pooled, w6, g6, b6, w7, b7)


def _bank_reshape(kmat, cout):
    return kmat.reshape(64, M, cout).transpose(1, 0, 2).reshape(M * 64, cout)


def _split3(v):
    hi = _bfx(v)
    r1 = v - hi
    mid = _bfx(r1)
    lo = r1 - mid
    return hi, mid, lo


def _prep(x, params):
    p = params
    xt = jnp.transpose(x, (0, 2, 1))
    xhi, xmid, xlo = _split3(xt)
    xx = jnp.sum(x * x, axis=1)
    ones1 = jnp.ones((B, N, 1), jnp.float32)
    xaug = jnp.concatenate(
        [xhi, xmid, xlo, -xx[..., None], ones1,
         jnp.zeros((B, N, 5), jnp.float32)], axis=2)
    xaug_t = jnp.transpose(jnp.concatenate(
        [xhi, ones1, -xx[..., None], jnp.zeros((B, N, 11), jnp.float32)],
        axis=2), (0, 2, 1))

    sn_params = []
    for nm in ("sn2", "sn3", "sn4"):
        w1t = jnp.pad(_bfx(p[nm + "_w1"]).T, ((0, 2), (0, 0)))
        sn_params.append((w1t, p[nm + "_g1"].reshape(1, 16),
                          p[nm + "_b1"].reshape(1, 16),
                          _bfx(p[nm + "_w2"].T), p[nm + "_b2"].reshape(1, M)))

    wconv = jnp.pad(_bfx(p["conv1_w"]).T, ((0, 13), (0, 0)))
    conv1_params = (wconv, p["bn1_g"].reshape(1, 64),
                    p["bn1_b"].reshape(1, 64))
    return xaug, xaug_t, sn_params, conv1_params


def kernel(x, params):
    p = params
    xaug, xaug_t, sn_params, conv1_params = _prep(x, params)
    fidx, s2, s3, s4, h1 = _knn_scores(xaug, xaug_t, sn_params, conv1_params)
    fidx_flat = fidx.reshape(-1)

    def bank(nm, bnm, cout):
        return (_bfx(_bank_reshape(p[nm], cout)),
                p[bnm + "_g"].reshape(1, cout), p[bnm + "_b"].reshape(1, cout))

    wr2, g2v, b2v = bank("matrice2", "bn2", 64)
    wr3, g3v, b3v = bank("matrice3", "bn3", 64)
    wr4, g4v, b4v = bank("matrice4", "bn4", 128)
    w5 = _bfx(p["conv5_w"].T)
    g5 = p["bn5_g"].reshape(1, 1024)
    b5 = p["bn5_b"].reshape(1, 1024)

    gat2 = _sc_gather(h1.reshape(B * N, 64), fidx_flat).reshape(B, N, KNN * 64)
    h2 = _layer(gat2, s2, wr2, g2v, b2v, 64)
    gat3 = _sc_gather(h2.reshape(B * N, 64), fidx_flat).reshape(B, N, KNN * 64)
    h3 = _layer(gat3, s3, wr3, g3v, b3v, 64)
    gat4 = _sc_gather(h3.reshape(B * N, 64), fidx_flat).reshape(B, N, KNN * 64)
    pooled = _layer4_pool(gat4, s4, wr4, g4v, b4v, w5, g5, b5)

    w6 = _bfx(p["linear1_w"].T)
    g6 = p["bn6_g"].reshape(1, 512)
    b6 = p["bn6_b"].reshape(1, 512)
    w7 = _bfx(p["linear2_w"].T)
    b7 = p["linear2_b"].reshape(1, 40)
    return _head(pooled, w6, g6, b6, w7, b7)


# speedup vs baseline: 29.8715x; 1.2247x over previous
import functools

import jax
import jax.numpy as jnp
import numpy as np
from jax import lax
from jax.experimental import pallas as pl
from jax.experimental.pallas import tpu as pltpu
from jax.experimental.pallas import tpu_sc as plsc

B, N, KNN, M = 8, 1024, 20, 8
EPS = 1e-5
INV_STD = np.float32(1.0 / np.sqrt(1.0 + EPS))
HI = jax.lax.Precision.HIGHEST

R = 256
NB = 256
NCORES, NSUBCORES = 2, 16
NW = NCORES * NSUBCORES
ROWS = B * N * KNN
RPW = ROWS // NW
CHUNK = 1280
NCHUNK = RPW // CHUNK


def _bf(v):
    return v.astype(jnp.bfloat16).astype(jnp.float32)


def _bfx(v):
    return lax.reduce_precision(v, exponent_bits=8, mantissa_bits=7)


def _knn_scores_kernel(xaug_row, xaug_full, xaug_t,
                       w1t2, g12, b12, w2t2, b22,
                       w1t3, g13, b13, w2t3, b23,
                       w1t4, g14, b14, w2t4, b24,
                       wconv, gc, bc,
                       fidx_ref, s2_ref, s3_ref, s4_ref, h1_ref):
    b = pl.program_id(0)
    a_r = xaug_row[0]
    full = xaug_full[0]
    xt_t = xaug_t[0]

    acc = a_r[:, 0:1] * xt_t[0:1, :]
    for d in (1, 2):
        acc = acc + a_r[:, d:d + 1] * xt_t[d:d + 1, :]
    pd = (a_r[:, 9:10] + (acc + acc)) + xt_t[4:5, :]

    ctr = (a_r[:, 0:3] + a_r[:, 3:6]) + a_r[:, 6:9]

    iota = lax.broadcasted_iota(jnp.int32, (R, N), 1)

    cur = pd
    idx_cols = []
    s2_cols, s3_cols, s4_cols = [], [], []
    for _ in range(KNN):
        v = jnp.max(cur, axis=1, keepdims=True)
        eqm = cur == v
        a = jnp.min(jnp.where(eqm, iota, jnp.int32(2**30)), axis=1,
                    keepdims=True)
        sel = iota == a
        onehot = sel.astype(jnp.float32)
        cur = jnp.where(sel, -jnp.inf, cur)
        idx_cols.append(a)
        nbrp = jnp.dot(onehot, full)
        nbr = (nbrp[:, 0:3] + nbrp[:, 3:6]) + nbrp[:, 6:9]
        diff = nbr - ctr
        xyzb = jnp.concatenate(
            [_bf(diff), _bf(nbr), jnp.zeros((R, 2), jnp.float32)], axis=1)
        for w1t, g1, b1, w2t, b2, cols in (
            (w1t2, g12, b12, w2t2, b22, s2_cols),
            (w1t3, g13, b13, w2t3, b23, s3_cols),
            (w1t4, g14, b14, w2t4, b24, s4_cols),
        ):
            z = jnp.dot(xyzb, w1t[...])
            act = jnp.maximum((z * INV_STD) * g1[...] + b1[...], 0.0)
            logits = jnp.dot(_bf(act), w2t[...]) + b2[...]
            e = jnp.exp(logits - jnp.max(logits, axis=1, keepdims=True))
            cols.append(e / jnp.sum(e, axis=1, keepdims=True))

    fidx_ref[0] = jnp.concatenate(idx_cols, axis=1) + b * N
    s2_ref[0] = jnp.concatenate(s2_cols, axis=1)
    s3_ref[0] = jnp.concatenate(s3_cols, axis=1)
    s4_ref[0] = jnp.concatenate(s4_cols, axis=1)
    zc = jnp.dot(a_r, wconv[...])
    h1_ref[0] = _bf(jnp.maximum((zc * INV_STD) * gc[...] + bc[...], 0.0))


def _knn_scores(xaug, xaug_t, sn_params, conv1_params):
    grid = (B, N // R)
    full_spec = pl.BlockSpec((1, N, 16), lambda b, r: (b, 0, 0))
    row_spec = pl.BlockSpec((1, R, 16), lambda b, r: (b, r, 0))
    t_spec = pl.BlockSpec((1, 16, N), lambda b, r: (b, 0, 0))

    def pspec(shape):
        return pl.BlockSpec(shape, lambda b, r: tuple(0 for _ in shape))

    in_specs = [row_spec, full_spec, t_spec]
    p_args = []
    for w1t, g1, b1, w2t, b2 in sn_params:
        p_args += [w1t, g1, b1, w2t, b2]
        in_specs += [pspec((8, 16)), pspec((1, 16)), pspec((1, 16)),
                     pspec((16, M)), pspec((1, M))]
    p_args += list(conv1_params)
    in_specs += [pspec((16, 64)), pspec((1, 64)), pspec((1, 64))]

    out_shape = [
        jax.ShapeDtypeStruct((B, N, KNN), jnp.int32),
        jax.ShapeDtypeStruct((B, N, KNN * M), jnp.float32),
        jax.ShapeDtypeStruct((B, N, KNN * M), jnp.float32),
        jax.ShapeDtypeStruct((B, N, KNN * M), jnp.float32),
        jax.ShapeDtypeStruct((B, N, 64), jnp.float32),
    ]
    out_specs = [
        pl.BlockSpec((1, R, KNN), lambda b, r: (b, r, 0)),
        pl.BlockSpec((1, R, KNN * M), lambda b, r: (b, r, 0)),
        pl.BlockSpec((1, R, KNN * M), lambda b, r: (b, r, 0)),
        pl.BlockSpec((1, R, KNN * M), lambda b, r: (b, r, 0)),
        pl.BlockSpec((1, R, 64), lambda b, r: (b, r, 0)),
    ]
    return pl.pallas_call(
        _knn_scores_kernel, grid=grid, in_specs=in_specs,
        out_specs=out_specs, out_shape=out_shape,
    )(xaug, xaug, xaug_t, *p_args)


def _sc_gather(table, fidx):
    mesh = plsc.VectorSubcoreMesh(core_axis_name="c", subcore_axis_name="s")

    @functools.partial(
        pl.kernel, mesh=mesh,
        compiler_params=pltpu.CompilerParams(use_tc_tiling_on_sc=False),
        out_type=jax.ShapeDtypeStruct((ROWS, 64), jnp.float32),
        scratch_types=[
            pltpu.VMEM((CHUNK,), jnp.int32),
            pltpu.VMEM((CHUNK, 64), jnp.float32),
            pltpu.SemaphoreType.DMA,
        ],
    )
    def gather_k(table_hbm, idx_hbm, out_hbm, idx_v, rows_v, sem):
        wid = lax.axis_index("s") * NCORES + lax.axis_index("c")
        base = wid * RPW
        for c in range(NCHUNK):
            off = base + c * CHUNK
            pltpu.sync_copy(idx_hbm.at[pl.ds(off, CHUNK)], idx_v)
            pltpu.async_copy(table_hbm.at[idx_v], rows_v, sem).wait()
            pltpu.sync_copy(rows_v, out_hbm.at[pl.ds(off, CHUNK)])

    return gather_k(table, fidx)


def _combine(g, s):
    parts = []
    for m in range(M):
        acc = jnp.zeros((NB, 64), jnp.float32)
        for k in range(KNN):
            acc = acc + s[:, k * M + m:k * M + m + 1] * g[:, k * 64:(k + 1) * 64]
        parts.append(acc)
    return jnp.concatenate(parts, axis=1)


def _dot_split(t, w):
    thi = _bf(t)
    tmid = _bf(t - thi)
    return jnp.dot(thi, w) + jnp.dot(tmid, w)


def _layer_kernel(g_ref, s_ref, wr, gv, bv, out_ref):
    t = _combine(g_ref[0], s_ref[0])
    z = _dot_split(t, wr[...])
    out_ref[0] = _bf(jnp.maximum((z * INV_STD) * gv[...] + bv[...], 0.0))


def _layer(g, s, wr, gv, bv, cout):
    grid = (B, N // NB)
    return pl.pallas_call(
        _layer_kernel, grid=grid,
        in_specs=[
            pl.BlockSpec((1, NB, KNN * 64), lambda b, r: (b, r, 0)),
            pl.BlockSpec((1, NB, KNN * M), lambda b, r: (b, r, 0)),
            pl.BlockSpec((M * 64, cout), lambda b, r: (0, 0)),
            pl.BlockSpec((1, cout), lambda b, r: (0, 0)),
            pl.BlockSpec((1, cout), lambda b, r: (0, 0)),
        ],
        out_specs=pl.BlockSpec((1, NB, cout), lambda b, r: (b, r, 0)),
        out_shape=jax.ShapeDtypeStruct((B, N, cout), jnp.float32),
    )(g, s, wr, gv, bv)


def _layer4_pool_kernel(g_ref, s_ref, wr, gv, bv, w5, g5, b5, out_ref):
    t = _combine(g_ref[0], s_ref[0])
    z = _dot_split(t, wr[...])
    h4 = jnp.maximum((z * INV_STD) * gv[...] + bv[...], 0.0)
    z5 = jnp.dot(_bf(h4), w5[...])
    z5 = jnp.maximum((z5 * INV_STD) * g5[...] + b5[...], 0.0)
    pm = jnp.max(z5, axis=0, keepdims=True)

    @pl.when(pl.program_id(1) == 0)
    def _():
        out_ref[0] = pm

    @pl.when(pl.program_id(1) != 0)
    def _():
        out_ref[0] = jnp.maximum(out_ref[0], pm)


def _layer4_pool(g, s, wr, gv, bv, w5, g5, b5):
    grid = (B, N // NB)
    return pl.pallas_call(
        _layer4_pool_kernel, grid=grid,
        in_specs=[
            pl.BlockSpec((1, NB, KNN * 64), lambda b, r: (b, r, 0)),
            pl.BlockSpec((1, NB, KNN * M), lambda b, r: (b, r, 0)),
            pl.BlockSpec((M * 64, 128), lambda b, r: (0, 0)),
            pl.BlockSpec((1, 128), lambda b, r: (0, 0)),
            pl.BlockSpec((1, 128), lambda b, r: (0, 0)),
            pl.BlockSpec((128, 1024), lambda b, r: (0, 0)),
            pl.BlockSpec((1, 1024), lambda b, r: (0, 0)),
            pl.BlockSpec((1, 1024), lambda b, r: (0, 0)),
        ],
        out_specs=pl.BlockSpec((1, 1, 1024), lambda b, r: (b, 0, 0)),
        out_shape=jax.ShapeDtypeStruct((B, 1, 1024), jnp.float32),
    )(g, s, wr, gv, bv, w5, g5, b5).reshape(B, 1024)


def _head_kernel(p_ref, w6, g6, b6, w7, b7, out_ref):
    z = jnp.dot(_bf(p_ref[...]), w6[...])
    h = jnp.maximum((z * INV_STD) * g6[...] + b6[...], 0.0)
    out_ref[...] = jnp.dot(_bf(h), w7[...]) + b7[...]


def _head(pooled, w6, g6, b6, w7, b7):
    return pl.pallas_call(
        _head_kernel,
        out_shape=jax.ShapeDtypeStruct((B, 40), jnp.float32),
    )(pooled, w6, g6, b6, w7, b7)


def _bank_reshape(kmat, cout):
    return kmat.reshape(64, M, cout).transpose(1, 0, 2).reshape(M * 64, cout)


def _split3(v):
    hi = _bfx(v)
    r1 = v - hi
    mid = _bfx(r1)
    lo = r1 - mid
    return hi, mid, lo


def _prep(x, params):
    p = params
    xt = jnp.transpose(x, (0, 2, 1))
    xhi, xmid, xlo = _split3(xt)
    xx = jnp.sum(x * x, axis=1)
    ones1 = jnp.ones((B, N, 1), jnp.float32)
    xaug = jnp.concatenate(
        [xhi, xmid, xlo, -xx[..., None], ones1,
         jnp.zeros((B, N, 5), jnp.float32)], axis=2)
    xaug_t = jnp.transpose(jnp.concatenate(
        [xhi, ones1, -xx[..., None], jnp.zeros((B, N, 11), jnp.float32)],
        axis=2), (0, 2, 1))

    sn_params = []
    for nm in ("sn2", "sn3", "sn4"):
        w1t = jnp.pad(_bfx(p[nm + "_w1"]).T, ((0, 2), (0, 0)))
        sn_params.append((w1t, p[nm + "_g1"].reshape(1, 16),
                          p[nm + "_b1"].reshape(1, 16),
                          _bfx(p[nm + "_w2"].T), p[nm + "_b2"].reshape(1, M)))

    wconv = jnp.pad(_bfx(p["conv1_w"]).T, ((0, 13), (0, 0)))
    conv1_params = (wconv, p["bn1_g"].reshape(1, 64),
                    p["bn1_b"].reshape(1, 64))
    return xaug, xaug_t, sn_params, conv1_params


def kernel(x, params):
    p = params
    xaug, xaug_t, sn_params, conv1_params = _prep(x, params)
    fidx, s2, s3, s4, h1 = _knn_scores(xaug, xaug_t, sn_params, conv1_params)
    fidx_flat = fidx.reshape(-1)

    def bank(nm, bnm, cout):
        return (_bfx(_bank_reshape(p[nm], cout)),
                p[bnm + "_g"].reshape(1, cout), p[bnm + "_b"].reshape(1, cout))

    wr2, g2v, b2v = bank("matrice2", "bn2", 64)
    wr3, g3v, b3v = bank("matrice3", "bn3", 64)
    wr4, g4v, b4v = bank("matrice4", "bn4", 128)
    w5 = _bfx(p["conv5_w"].T)
    g5 = p["bn5_g"].reshape(1, 1024)
    b5 = p["bn5_b"].reshape(1, 1024)

    gat2 = _sc_gather(h1.reshape(B * N, 64), fidx_flat).reshape(B, N, KNN * 64)
    h2 = _layer(gat2, s2, wr2, g2v, b2v, 64)
    gat3 = _sc_gather(h2.reshape(B * N, 64), fidx_flat).reshape(B, N, KNN * 64)
    h3 = _layer(gat3, s3, wr3, g3v, b3v, 64)
    gat4 = _sc_gather(h3.reshape(B * N, 64), fidx_flat).reshape(B, N, KNN * 64)
    pooled = _layer4_pool(gat4, s4, wr4, g4v, b4v, w5, g5, b5)

    w6 = _bfx(p["linear1_w"].T)
    g6 = p["bn6_g"].reshape(1, 512)
    b6 = p["bn6_b"].reshape(1, 512)
    w7 = _bfx(p["linear2_w"].T)
    b7 = p["linear2_b"].reshape(1, 40)
    return _head(pooled, w6, g6, b6, w7, b7)


# speedup vs baseline: 75.1906x; 2.5171x over previous
import functools

import jax
import jax.numpy as jnp
import numpy as np
from jax import lax
from jax.experimental import pallas as pl
from jax.experimental.pallas import tpu as pltpu
from jax.experimental.pallas import tpu_sc as plsc

B, N, KNN, M = 8, 1024, 20, 8
EPS = 1e-5
INV_STD = np.float32(1.0 / np.sqrt(1.0 + EPS))
HI = jax.lax.Precision.HIGHEST

R = 256
NB = 256
NCORES, NSUBCORES = 2, 16
NW = NCORES * NSUBCORES
ROWS = B * N * KNN
RPW = ROWS // NW
CHUNK = 1280
NCHUNK = RPW // CHUNK


def _bf(v):
    return v.astype(jnp.bfloat16).astype(jnp.float32)


def _bfx(v):
    return lax.reduce_precision(v, exponent_bits=8, mantissa_bits=7)


def _knn_scores_kernel(xaug_row, xaug_full, xaug_t,
                       w1bd2, g12, b12, w2bd2, b22,
                       w1bd3, g13, b13, w2bd3, b23,
                       w1bd4, g14, b14, w2bd4, b24,
                       wconv, gc, bc, bdsum,
                       fidx_ref, s2_ref, s3_ref, s4_ref, h1_ref):
    b = pl.program_id(0)
    a_r = xaug_row[0]
    full = xaug_full[0]
    xt_t = xaug_t[0]

    acc = a_r[:, 0:1] * xt_t[0:1, :]
    for d in (1, 2):
        acc = acc + a_r[:, d:d + 1] * xt_t[d:d + 1, :]
    pd = (a_r[:, 9:10] + (acc + acc)) + xt_t[4:5, :]

    ctr = (a_r[:, 0:3] + a_r[:, 3:6]) + a_r[:, 6:9]

    iota = lax.broadcasted_iota(jnp.int32, (R, N), 1)

    cur = pd
    idx_cols = []
    xyz_cols = []
    for _ in range(KNN):
        v = jnp.max(cur, axis=1, keepdims=True)
        eqm = cur == v
        a = jnp.min(jnp.where(eqm, iota, jnp.int32(2**30)), axis=1,
                    keepdims=True)
        sel = iota == a
        onehot = sel.astype(jnp.float32)
        cur = jnp.where(sel, -jnp.inf, cur)
        idx_cols.append(a)
        nbrp = jnp.dot(onehot, full)
        nbr = (nbrp[:, 0:3] + nbrp[:, 3:6]) + nbrp[:, 6:9]
        diff = nbr - ctr
        xyz_cols.append(jnp.concatenate(
            [_bf(diff), _bf(nbr), jnp.zeros((R, 2), jnp.float32)], axis=1))

    fidx_ref[0] = jnp.concatenate(idx_cols, axis=1) + b * N
    xk = jnp.concatenate(xyz_cols, axis=1)
    for w1bd, g1, b1, w2bd, b2, s_out in (
        (w1bd2, g12, b12, w2bd2, b22, s2_ref),
        (w1bd3, g13, b13, w2bd3, b23, s3_ref),
        (w1bd4, g14, b14, w2bd4, b24, s4_ref),
    ):
        z = jnp.dot(xk, w1bd[...])
        act = jnp.maximum((z * INV_STD) * g1[...] + b1[...], 0.0)
        logits = jnp.dot(_bf(act), w2bd[...]) + b2[...]
        e = jnp.exp(logits)
        ehi = _bf(e)
        emid = _bf(e - ehi)
        denom = jnp.dot(ehi, bdsum[...]) + jnp.dot(emid, bdsum[...])
        s_out[0] = e / denom
    zc = jnp.dot(a_r, wconv[...])
    h1_ref[0] = _bf(jnp.maximum((zc * INV_STD) * gc[...] + bc[...], 0.0))


def _knn_scores(xaug, xaug_t, sn_params, conv1_params, bdsum):
    grid = (B, N // R)
    full_spec = pl.BlockSpec((1, N, 16), lambda b, r: (b, 0, 0))
    row_spec = pl.BlockSpec((1, R, 16), lambda b, r: (b, r, 0))
    t_spec = pl.BlockSpec((1, 16, N), lambda b, r: (b, 0, 0))

    def pspec(shape):
        return pl.BlockSpec(shape, lambda b, r: tuple(0 for _ in shape))

    in_specs = [row_spec, full_spec, t_spec]
    p_args = []
    for w1bd, g1, b1, w2bd, b2 in sn_params:
        p_args += [w1bd, g1, b1, w2bd, b2]
        in_specs += [pspec((8 * KNN, 16 * KNN)), pspec((1, 16 * KNN)),
                     pspec((1, 16 * KNN)), pspec((16 * KNN, M * KNN)),
                     pspec((1, M * KNN))]
    p_args += list(conv1_params)
    in_specs += [pspec((16, 64)), pspec((1, 64)), pspec((1, 64)),
                 pspec((M * KNN, M * KNN))]

    out_shape = [
        jax.ShapeDtypeStruct((B, N, KNN), jnp.int32),
        jax.ShapeDtypeStruct((B, N, KNN * M), jnp.float32),
        jax.ShapeDtypeStruct((B, N, KNN * M), jnp.float32),
        jax.ShapeDtypeStruct((B, N, KNN * M), jnp.float32),
        jax.ShapeDtypeStruct((B, N, 64), jnp.float32),
    ]
    out_specs = [
        pl.BlockSpec((1, R, KNN), lambda b, r: (b, r, 0)),
        pl.BlockSpec((1, R, KNN * M), lambda b, r: (b, r, 0)),
        pl.BlockSpec((1, R, KNN * M), lambda b, r: (b, r, 0)),
        pl.BlockSpec((1, R, KNN * M), lambda b, r: (b, r, 0)),
        pl.BlockSpec((1, R, 64), lambda b, r: (b, r, 0)),
    ]
    return pl.pallas_call(
        _knn_scores_kernel, grid=grid, in_specs=in_specs,
        out_specs=out_specs, out_shape=out_shape,
    )(xaug, xaug, xaug_t, *p_args, bdsum)


def _sc_gather(table, fidx):
    mesh = plsc.VectorSubcoreMesh(core_axis_name="c", subcore_axis_name="s")

    @functools.partial(
        pl.kernel, mesh=mesh,
        compiler_params=pltpu.CompilerParams(use_tc_tiling_on_sc=False),
        out_type=jax.ShapeDtypeStruct((ROWS, 64), jnp.float32),
        scratch_types=[
            pltpu.VMEM((CHUNK,), jnp.int32),
            pltpu.VMEM((CHUNK, 64), jnp.float32),
            pltpu.SemaphoreType.DMA,
        ],
    )
    def gather_k(table_hbm, idx_hbm, out_hbm, idx_v, rows_v, sem):
        wid = lax.axis_index("s") * NCORES + lax.axis_index("c")
        base = wid * RPW
        for c in range(NCHUNK):
            off = base + c * CHUNK
            pltpu.sync_copy(idx_hbm.at[pl.ds(off, CHUNK)], idx_v)
            pltpu.async_copy(table_hbm.at[idx_v], rows_v, sem).wait()
            pltpu.sync_copy(rows_v, out_hbm.at[pl.ds(off, CHUNK)])

    return gather_k(table, fidx)


def _combine(g, s, e8):
    sb = _bf(s)
    accs = [jnp.zeros((NB, 128), jnp.float32) for _ in range(M // 2)]
    for k in range(KNN):
        srep = jnp.dot(sb[:, k * M:(k + 1) * M], e8[...])
        gk = g[:, k * 64:(k + 1) * 64]
        g2 = jnp.concatenate([gk, gk], axis=1)
        for p2 in range(M // 2):
            accs[p2] = accs[p2] + srep[:, p2 * 128:(p2 + 1) * 128] * g2
    return jnp.concatenate(accs, axis=1)


def _dot_split(t, w):
    thi = _bf(t)
    tmid = _bf(t - thi)
    return jnp.dot(thi, w) + jnp.dot(tmid, w)


def _layer_kernel(g_ref, s_ref, wr, gv, bv, e8, out_ref):
    t = _combine(g_ref[0], s_ref[0], e8)
    z = _dot_split(t, wr[...])
    out_ref[0] = _bf(jnp.maximum((z * INV_STD) * gv[...] + bv[...], 0.0))


def _layer(g, s, wr, gv, bv, e8, cout):
    grid = (B, N // NB)
    return pl.pallas_call(
        _layer_kernel, grid=grid,
        in_specs=[
            pl.BlockSpec((1, NB, KNN * 64), lambda b, r: (b, r, 0)),
            pl.BlockSpec((1, NB, KNN * M), lambda b, r: (b, r, 0)),
            pl.BlockSpec((M * 64, cout), lambda b, r: (0, 0)),
            pl.BlockSpec((1, cout), lambda b, r: (0, 0)),
            pl.BlockSpec((1, cout), lambda b, r: (0, 0)),
            pl.BlockSpec((M, M * 64), lambda b, r: (0, 0)),
        ],
        out_specs=pl.BlockSpec((1, NB, cout), lambda b, r: (b, r, 0)),
        out_shape=jax.ShapeDtypeStruct((B, N, cout), jnp.float32),
    )(g, s, wr, gv, bv, e8)


def _layer4_pool_kernel(g_ref, s_ref, wr, gv, bv, w5, g5, b5, e8, out_ref):
    t = _combine(g_ref[0], s_ref[0], e8)
    z = _dot_split(t, wr[...])
    h4 = jnp.maximum((z * INV_STD) * gv[...] + bv[...], 0.0)
    z5 = jnp.dot(_bf(h4), w5[...])
    z5 = jnp.maximum((z5 * INV_STD) * g5[...] + b5[...], 0.0)
    pm = jnp.max(z5, axis=0, keepdims=True)

    @pl.when(pl.program_id(1) == 0)
    def _():
        out_ref[0] = pm

    @pl.when(pl.program_id(1) != 0)
    def _():
        out_ref[0] = jnp.maximum(out_ref[0], pm)


def _layer4_pool(g, s, wr, gv, bv, w5, g5, b5, e8):
    grid = (B, N // NB)
    return pl.pallas_call(
        _layer4_pool_kernel, grid=grid,
        in_specs=[
            pl.BlockSpec((1, NB, KNN * 64), lambda b, r: (b, r, 0)),
            pl.BlockSpec((1, NB, KNN * M), lambda b, r: (b, r, 0)),
            pl.BlockSpec((M * 64, 128), lambda b, r: (0, 0)),
            pl.BlockSpec((1, 128), lambda b, r: (0, 0)),
            pl.BlockSpec((1, 128), lambda b, r: (0, 0)),
            pl.BlockSpec((128, 1024), lambda b, r: (0, 0)),
            pl.BlockSpec((1, 1024), lambda b, r: (0, 0)),
            pl.BlockSpec((1, 1024), lambda b, r: (0, 0)),
            pl.BlockSpec((M, M * 64), lambda b, r: (0, 0)),
        ],
        out_specs=pl.BlockSpec((1, 1, 1024), lambda b, r: (b, 0, 0)),
        out_shape=jax.ShapeDtypeStruct((B, 1, 1024), jnp.float32),
    )(g, s, wr, gv, bv, w5, g5, b5, e8).reshape(B, 1024)


def _head_kernel(p_ref, w6, g6, b6, w7, b7, out_ref):
    z = jnp.dot(_bf(p_ref[...]), w6[...])
    h = jnp.maximum((z * INV_STD) * g6[...] + b6[...], 0.0)
    out_ref[...] = jnp.dot(_bf(h), w7[...]) + b7[...]


def _head(pooled, w6, g6, b6, w7, b7):
    return pl.pallas_call(
        _head_kernel,
        out_shape=jax.ShapeDtypeStruct((B, 40), jnp.float32),
    )(pooled, w6, g6, b6, w7, b7)


def _bank_reshape(kmat, cout):
    return kmat.reshape(64, M, cout).transpose(1, 0, 2).reshape(M * 64, cout)


def _split3(v):
    hi = _bfx(v)
    r1 = v - hi
    mid = _bfx(r1)
    lo = r1 - mid
    return hi, mid, lo


def _prep(x, params):
    p = params
    xt = jnp.transpose(x, (0, 2, 1))
    xhi, xmid, xlo = _split3(xt)
    xx = jnp.sum(x * x, axis=1)
    ones1 = jnp.ones((B, N, 1), jnp.float32)
    xaug = jnp.concatenate(
        [xhi, xmid, xlo, -xx[..., None], ones1,
         jnp.zeros((B, N, 5), jnp.float32)], axis=2)
    xaug_t = jnp.transpose(jnp.concatenate(
        [xhi, ones1, -xx[..., None], jnp.zeros((B, N, 11), jnp.float32)],
        axis=2), (0, 2, 1))

    eye20 = jnp.eye(KNN, dtype=jnp.float32)
    sn_params = []
    for nm in ("sn2", "sn3", "sn4"):
        w1t = jnp.pad(_bfx(p[nm + "_w1"]).T, ((0, 2), (0, 0)))
        w1bd = jnp.kron(eye20, w1t)
        w2bd = jnp.kron(eye20, _bfx(p[nm + "_w2"].T))
        sn_params.append((w1bd, jnp.tile(p[nm + "_g1"], KNN).reshape(1, -1),
                          jnp.tile(p[nm + "_b1"], KNN).reshape(1, -1),
                          w2bd, jnp.tile(p[nm + "_b2"], KNN).reshape(1, -1)))
    bdsum = jnp.kron(eye20, jnp.ones((M, M), jnp.float32))
    e8 = jnp.kron(jnp.eye(M, dtype=jnp.float32), jnp.ones((1, 64), jnp.float32))

    wconv = jnp.pad(_bfx(p["conv1_w"]).T, ((0, 13), (0, 0)))
    conv1_params = (wconv, p["bn1_g"].reshape(1, 64),
                    p["bn1_b"].reshape(1, 64))
    return xaug, xaug_t, sn_params, conv1_params, bdsum, e8


def kernel(x, params):
    p = params
    xaug, xaug_t, sn_params, conv1_params, bdsum, e8 = _prep(x, params)
    fidx, s2, s3, s4, h1 = _knn_scores(xaug, xaug_t, sn_params, conv1_params,
                                       bdsum)
    fidx_flat = fidx.reshape(-1)

    def bank(nm, bnm, cout):
        return (_bfx(_bank_reshape(p[nm], cout)),
                p[bnm + "_g"].reshape(1, cout), p[bnm + "_b"].reshape(1, cout))

    wr2, g2v, b2v = bank("matrice2", "bn2", 64)
    wr3, g3v, b3v = bank("matrice3", "bn3", 64)
    wr4, g4v, b4v = bank("matrice4", "bn4", 128)
    w5 = _bfx(p["conv5_w"].T)
    g5 = p["bn5_g"].reshape(1, 1024)
    b5 = p["bn5_b"].reshape(1, 1024)

    gat2 = _sc_gather(h1.reshape(B * N, 64), fidx_flat).reshape(B, N, KNN * 64)
    h2 = _layer(gat2, s2, wr2, g2v, b2v, e8, 64)
    gat3 = _sc_gather(h2.reshape(B * N, 64), fidx_flat).reshape(B, N, KNN * 64)
    h3 = _layer(gat3, s3, wr3, g3v, b3v, e8, 64)
    gat4 = _sc_gather(h3.reshape(B * N, 64), fidx_flat).reshape(B, N, KNN * 64)
    pooled = _layer4_pool(gat4, s4, wr4, g4v, b4v, w5, g5, b5, e8)

    w6 = _bfx(p["linear1_w"].T)
    g6 = p["bn6_g"].reshape(1, 512)
    b6 = p["bn6_b"].reshape(1, 512)
    w7 = _bfx(p["linear2_w"].T)
    b7 = p["linear2_b"].reshape(1, 40)
    return _head(pooled, w6, g6, b6, w7, b7)
